# Initial kernel scaffold; baseline (speedup 1.0000x reference)
import jax
import jax.numpy as jnp
from jax.experimental import pallas as pl


def kernel(x, table):
    raise NotImplementedError("write your pallas kernel here")


# speedup vs baseline: 2.9790x; 2.9790x over previous
import functools

import jax
import jax.numpy as jnp
from jax import lax
from jax.experimental import pallas as pl
from jax.experimental.pallas import tpu as pltpu
from jax.experimental.pallas import tpu_sc as plsc

NUM_CLASSES = 100000
EMBED_DIM = 128

NC = 2
NS = 16
NW = NC * NS

C = 128
B = 4096 * 50
CPW = B // (NW * C)


def _sc_gather(idx2d, table):
    mesh = plsc.VectorSubcoreMesh(
        core_axis_name="c", subcore_axis_name="s", num_cores=NC, num_subcores=NS
    )

    @functools.partial(
        pl.kernel,
        out_type=jax.ShapeDtypeStruct((B, EMBED_DIM), jnp.float32),
        mesh=mesh,
        scratch_types=[
            pltpu.VMEM((CPW, C), jnp.int32),
            pltpu.VMEM((C, EMBED_DIM), jnp.float32),
            pltpu.SemaphoreType.DMA,
            pltpu.SemaphoreType.DMA,
        ],
    )
    def k(idx_hbm, table_hbm, out_hbm, idx_v, rows_v, gsem, ssem):
        wid = lax.axis_index("s") * NC + lax.axis_index("c")
        pltpu.sync_copy(idx_hbm.at[wid], idx_v)
        base = wid * CPW * C

        def body(j, carry):
            pltpu.async_copy(table_hbm.at[idx_v.at[j]], rows_v, gsem).wait()
            pltpu.async_copy(rows_v, out_hbm.at[pl.ds(base + j * C, C)], ssem).wait()
            return carry

        lax.fori_loop(0, CPW, body, 0)

    return k(idx2d, table)


def kernel(x, table):
    idx2d = x.astype(jnp.int32).reshape(NW, CPW, C)
    out = _sc_gather(idx2d, table)
    return out.reshape(x.shape[0], x.shape[1], EMBED_DIM)


# speedup vs baseline: 3.3128x; 1.1120x over previous
import functools

import jax
import jax.numpy as jnp
from jax import lax
from jax.experimental import pallas as pl
from jax.experimental.pallas import tpu as pltpu
from jax.experimental.pallas import tpu_sc as plsc

NUM_CLASSES = 100000
EMBED_DIM = 128

NC = 2
NS = 16
NW = NC * NS

C = 128
B = 4096 * 50
CPW = B // (NW * C)
NBUF = 5
NITER = CPW // NBUF


def _sc_gather(idx2d, table):
    mesh = plsc.VectorSubcoreMesh(
        core_axis_name="c", subcore_axis_name="s", num_cores=NC, num_subcores=NS
    )

    @functools.partial(
        pl.kernel,
        out_type=jax.ShapeDtypeStruct((B, EMBED_DIM), jnp.float32),
        mesh=mesh,
        scratch_types=[
            pltpu.VMEM((CPW, C), jnp.int32),
            pltpu.VMEM((NBUF, C, EMBED_DIM), jnp.float32),
        ]
        + [pltpu.SemaphoreType.DMA] * (2 * NBUF),
    )
    def k(idx_hbm, table_hbm, out_hbm, idx_v, rows_v, *sems):
        gsems, ssems = sems[:NBUF], sems[NBUF:]
        wid = lax.axis_index("s") * NC + lax.axis_index("c")
        pltpu.sync_copy(idx_hbm.at[wid], idx_v)
        base = wid * CPW * C

        def gather(j, b):
            return pltpu.make_async_copy(
                table_hbm.at[idx_v.at[j]], rows_v.at[b], gsems[b]
            )

        def store(j, b):
            return pltpu.make_async_copy(
                rows_v.at[b], out_hbm.at[pl.ds(base + j * C, C)], ssems[b]
            )

        for b in range(NBUF):
            gather(b, b).start()

        def body(g, carry):
            for b in range(NBUF):
                j = g * NBUF + b
                gather(j, b).wait()
                store(j, b).start()

            @pl.when(g < NITER - 1)
            def _():
                for b in range(NBUF):
                    j = (g + 1) * NBUF + b
                    store(j - NBUF, b).wait()
                    gather(j, b).start()

            return carry

        lax.fori_loop(0, NITER, body, 0)
        for b in range(NBUF):
            store(0, b).wait()

    return k(idx2d, table)


def kernel(x, table):
    idx2d = x.astype(jnp.int32).reshape(NW, CPW, C)
    out = _sc_gather(idx2d, table)
    return out.reshape(x.shape[0], x.shape[1], EMBED_DIM)


# speedup vs baseline: 3.3544x; 1.0126x over previous
import functools

import jax
import jax.numpy as jnp
from jax import lax
from jax.experimental import pallas as pl
from jax.experimental.pallas import tpu as pltpu
from jax.experimental.pallas import tpu_sc as plsc

NUM_CLASSES = 100000
EMBED_DIM = 128

NC = 2
NS = 16
NW = NC * NS

C = 128
B = 4096 * 50
CPW = B // (NW * C)
NBUF = 5
NITER = CPW // NBUF


def _sc_gather(idx2d, table):
    mesh = plsc.VectorSubcoreMesh(
        core_axis_name="c", subcore_axis_name="s", num_cores=NC, num_subcores=NS
    )

    @functools.partial(
        pl.kernel,
        out_type=jax.ShapeDtypeStruct((B, EMBED_DIM), jnp.float32),
        mesh=mesh,
        scratch_types=[
            pltpu.VMEM((CPW, C), jnp.int32),
            pltpu.VMEM((NBUF, C, EMBED_DIM), jnp.float32),
        ]
        + [pltpu.SemaphoreType.DMA] * (2 * NBUF),
    )
    def k(idx_hbm, table_hbm, out_hbm, idx_v, rows_v, *sems):
        gsems, ssems = sems[:NBUF], sems[NBUF:]
        wid = lax.axis_index("s") * NC + lax.axis_index("c")
        pltpu.sync_copy(idx_hbm.at[wid], idx_v)
        base = wid * CPW * C

        def gather(j, b):
            return pltpu.make_async_copy(
                table_hbm.at[idx_v.at[j]], rows_v.at[b], gsems[b]
            )

        def store(j, b):
            return pltpu.make_async_copy(
                rows_v.at[b], out_hbm.at[pl.ds(base + j * C, C)], ssems[b]
            )

        for b in range(NBUF - 1):
            gather(b, b).start()

        def body(g, carry):
            for b in range(NBUF):
                j = g * NBUF + b
                gather(j, b).wait()
                store(j, b).start()
                pb = (b - 1) % NBUF
                if b == 0:
                    @pl.when(g > 0)
                    def _():
                        store(j - 1, pb).wait()

                else:
                    store(j - 1, pb).wait()

                @pl.when(j + NBUF - 1 < CPW)
                def _():
                    gather(j + NBUF - 1, pb).start()

            return carry

        lax.fori_loop(0, NITER, body, 0)
        store(CPW - 1, (CPW - 1) % NBUF).wait()

    return k(idx2d, table)


def kernel(x, table):
    idx2d = x.astype(jnp.int32).reshape(NW, CPW, C)
    out = _sc_gather(idx2d, table)
    return out.reshape(x.shape[0], x.shape[1], EMBED_DIM)


# speedup vs baseline: 5.9721x; 1.7804x over previous
import functools

import jax
import jax.numpy as jnp
from jax import lax
from jax.experimental import pallas as pl
from jax.experimental.pallas import tpu as pltpu
from jax.experimental.pallas import tpu_sc as plsc

NUM_CLASSES = 100000
EMBED_DIM = 128
BATCH = 4096
SEQ = 50
SEQ_PAD = 64

NC = 2
NS = 16
NW = NC * NS

RPW = BATCH // NW
R = 4
CH = RPW // R
NBUF = 4
NITER = CH // NBUF


def _sc_gather(idx3, table):
    mesh = plsc.VectorSubcoreMesh(
        core_axis_name="c", subcore_axis_name="s", num_cores=NC, num_subcores=NS
    )

    @functools.partial(
        pl.kernel,
        out_type=jax.ShapeDtypeStruct((BATCH, SEQ, EMBED_DIM), jnp.float32),
        mesh=mesh,
        scratch_types=[
            pltpu.VMEM((RPW, SEQ_PAD), jnp.int32),
            pltpu.VMEM((NBUF, R, SEQ, EMBED_DIM), jnp.float32),
        ]
        + [pltpu.SemaphoreType.DMA] * (2 * NBUF),
    )
    def k(idx_hbm, table_hbm, out_hbm, idx_v, rows_v, *sems):
        gsems, ssems = sems[:NBUF], sems[NBUF:]
        wid = lax.axis_index("s") * NC + lax.axis_index("c")
        pltpu.sync_copy(idx_hbm.at[wid], idx_v)
        base = wid * RPW

        def gathers(j, b):
            return [
                pltpu.make_async_copy(
                    table_hbm.at[idx_v.at[j * R + rr, pl.ds(0, SEQ)]],
                    rows_v.at[b, rr],
                    gsems[b],
                )
                for rr in range(R)
            ]

        def store(j, b):
            return pltpu.make_async_copy(
                rows_v.at[b], out_hbm.at[pl.ds(base + j * R, R)], ssems[b]
            )

        for b in range(NBUF - 1):
            for cp in gathers(b, b):
                cp.start()

        def body(g, carry):
            for b in range(NBUF):
                j = g * NBUF + b
                pb = (b - 1) % NBUF
                for cp in gathers(j, b):
                    cp.wait()
                store(j, b).start()
                if b == 0:
                    @pl.when(g > 0)
                    def _():
                        store(j - 1, pb).wait()

                else:
                    store(j - 1, pb).wait()

                @pl.when(j + NBUF - 1 < CH)
                def _():
                    for cp in gathers(j + NBUF - 1, pb):
                        cp.start()

            return carry

        lax.fori_loop(0, NITER, body, 0)
        store(CH - 1, (CH - 1) % NBUF).wait()

    return k(idx3, table)


def kernel(x, table):
    idx = x.astype(jnp.int32).reshape(NW, RPW, SEQ)
    idx3 = jnp.pad(idx, ((0, 0), (0, 0), (0, SEQ_PAD - SEQ)))
    return _sc_gather(idx3, table)


# speedup vs baseline: 5.9732x; 1.0002x over previous
import functools

import jax
import jax.numpy as jnp
from jax import lax
from jax.experimental import pallas as pl
from jax.experimental.pallas import tpu as pltpu
from jax.experimental.pallas import tpu_sc as plsc

NUM_CLASSES = 100000
EMBED_DIM = 128
BATCH = 4096
SEQ = 50
SEQ_PAD = 64

NC = 2
NS = 16
NW = NC * NS

RPW = BATCH // NW
R = 4
CH = RPW // R
NBUF = 4
NITER = CH // NBUF


def _sc_gather(idx3, table):
    mesh = plsc.VectorSubcoreMesh(
        core_axis_name="c", subcore_axis_name="s", num_cores=NC, num_subcores=NS
    )

    @functools.partial(
        pl.kernel,
        out_type=jax.ShapeDtypeStruct((BATCH, SEQ, EMBED_DIM), jnp.float32),
        mesh=mesh,
        scratch_types=[
            pltpu.VMEM((RPW, SEQ_PAD), jnp.int32),
            pltpu.VMEM((NBUF, R, SEQ, EMBED_DIM), jnp.float32),
        ]
        + [pltpu.SemaphoreType.DMA] * (2 * NBUF),
        compiler_params=pltpu.CompilerParams(use_tc_tiling_on_sc=True),
    )
    def k(idx_hbm, table_hbm, out_hbm, idx_v, rows_v, *sems):
        gsems, ssems = sems[:NBUF], sems[NBUF:]
        wid = lax.axis_index("s") * NC + lax.axis_index("c")
        pltpu.sync_copy(idx_hbm.at[wid], idx_v)
        base = wid * RPW

        def gathers(j, b):
            return [
                pltpu.make_async_copy(
                    table_hbm.at[idx_v.at[j * R + rr, pl.ds(0, SEQ)]],
                    rows_v.at[b, rr],
                    gsems[b],
                )
                for rr in range(R)
            ]

        def store(j, b):
            return pltpu.make_async_copy(
                rows_v.at[b], out_hbm.at[pl.ds(base + j * R, R)], ssems[b]
            )

        for b in range(NBUF - 1):
            for cp in gathers(b, b):
                cp.start()

        def body(g, carry):
            for b in range(NBUF):
                j = g * NBUF + b
                pb = (b - 1) % NBUF
                for cp in gathers(j, b):
                    cp.wait()
                store(j, b).start()
                if b == 0:
                    @pl.when(g > 0)
                    def _():
                        store(j - 1, pb).wait()

                else:
                    store(j - 1, pb).wait()

                @pl.when(j + NBUF - 1 < CH)
                def _():
                    for cp in gathers(j + NBUF - 1, pb):
                        cp.start()

            return carry

        lax.fori_loop(0, NITER, body, 0)
        store(CH - 1, (CH - 1) % NBUF).wait()

    return k(idx3, table)


def kernel(x, table):
    idx = x.astype(jnp.int32).reshape(NW, RPW, SEQ)
    idx3 = jnp.pad(idx, ((0, 0), (0, 0), (0, SEQ_PAD - SEQ)))
    return _sc_gather(idx3, table)


# speedup vs baseline: 10.7491x; 1.7996x over previous
import functools

import jax
import jax.numpy as jnp
from jax import lax
from jax.experimental import pallas as pl
from jax.experimental.pallas import tpu as pltpu
from jax.experimental.pallas import tpu_sc as plsc

NUM_CLASSES = 100000
EMBED_DIM = 128
BATCH = 4096
SEQ = 50

NC = 2
NS = 16
NW = NC * NS

C = BATCH // NW
NBUF = 5
NITER = SEQ // NBUF


def _sc_gather(xt, table):
    mesh = plsc.VectorSubcoreMesh(
        core_axis_name="c", subcore_axis_name="s", num_cores=NC, num_subcores=NS
    )

    @functools.partial(
        pl.kernel,
        out_type=jax.ShapeDtypeStruct((SEQ, BATCH, EMBED_DIM), jnp.float32),
        mesh=mesh,
        scratch_types=[
            pltpu.VMEM((SEQ, C), jnp.int32),
            pltpu.VMEM((NBUF, C, EMBED_DIM), jnp.float32),
        ]
        + [pltpu.SemaphoreType.DMA] * (2 * NBUF),
    )
    def k(xt_hbm, table_hbm, out_hbm, idx_v, rows_v, *sems):
        gsems, ssems = sems[:NBUF], sems[NBUF:]
        wid = lax.axis_index("s") * NC + lax.axis_index("c")
        col0 = wid * C
        pltpu.sync_copy(xt_hbm.at[:, pl.ds(col0, C)], idx_v)

        def gather(t, b):
            return pltpu.make_async_copy(
                table_hbm.at[idx_v.at[t]], rows_v.at[b], gsems[b]
            )

        def store(t, b):
            return pltpu.make_async_copy(
                rows_v.at[b], out_hbm.at[t, pl.ds(col0, C)], ssems[b]
            )

        for b in range(NBUF - 1):
            gather(b, b).start()

        def body(g, carry):
            for b in range(NBUF):
                t = g * NBUF + b
                pb = (b - 1) % NBUF
                gather(t, b).wait()
                store(t, b).start()
                if b == 0:
                    @pl.when(g > 0)
                    def _():
                        store(t - 1, pb).wait()

                else:
                    store(t - 1, pb).wait()

                @pl.when(t + NBUF - 1 < SEQ)
                def _():
                    gather(t + NBUF - 1, pb).start()

            return carry

        lax.fori_loop(0, NITER, body, 0)
        store(SEQ - 1, (SEQ - 1) % NBUF).wait()

    return k(xt, table)


def kernel(x, table):
    xt = x.astype(jnp.int32).T
    out = _sc_gather(xt, table)
    return out.transpose(1, 0, 2)


# speedup vs baseline: 10.7871x; 1.0035x over previous
import functools

import jax
import jax.numpy as jnp
from jax import lax
from jax.experimental import pallas as pl
from jax.experimental.pallas import tpu as pltpu
from jax.experimental.pallas import tpu_sc as plsc

NUM_CLASSES = 100000
EMBED_DIM = 128
BATCH = 4096
SEQ = 50

NC = 2
NS = 16
NW = NC * NS

C = BATCH // NW
NBUF = 5
NITER = SEQ // NBUF


def _sc_gather(xt, table):
    mesh = plsc.VectorSubcoreMesh(
        core_axis_name="c", subcore_axis_name="s", num_cores=NC, num_subcores=NS
    )

    @functools.partial(
        pl.kernel,
        out_type=jax.ShapeDtypeStruct((SEQ, BATCH, EMBED_DIM), jnp.float32),
        mesh=mesh,
        scratch_types=[
            pltpu.VMEM((SEQ, C), jnp.int32),
            pltpu.VMEM((NBUF, C, EMBED_DIM), jnp.float32),
        ]
        + [pltpu.SemaphoreType.DMA] * (2 * NBUF),
        compiler_params=pltpu.CompilerParams(
            disable_bounds_checks=True, disable_semaphore_checks=True
        ),
    )
    def k(xt_hbm, table_hbm, out_hbm, idx_v, rows_v, *sems):
        gsems, ssems = sems[:NBUF], sems[NBUF:]
        wid = lax.axis_index("s") * NC + lax.axis_index("c")
        col0 = wid * C
        pltpu.sync_copy(xt_hbm.at[:, pl.ds(col0, C)], idx_v)

        def gather(t, b):
            return pltpu.make_async_copy(
                table_hbm.at[idx_v.at[t]], rows_v.at[b], gsems[b]
            )

        def store(t, b):
            return pltpu.make_async_copy(
                rows_v.at[b], out_hbm.at[t, pl.ds(col0, C)], ssems[b]
            )

        for b in range(NBUF - 1):
            gather(b, b).start()

        def body(g, carry):
            for b in range(NBUF):
                t = g * NBUF + b
                pb = (b - 1) % NBUF
                gather(t, b).wait()
                store(t, b).start()
                if b == 0:
                    @pl.when(g > 0)
                    def _():
                        store(t - 1, pb).wait()

                else:
                    store(t - 1, pb).wait()

                @pl.when(t + NBUF - 1 < SEQ)
                def _():
                    gather(t + NBUF - 1, pb).start()

            return carry

        lax.fori_loop(0, NITER, body, 0)
        store(SEQ - 1, (SEQ - 1) % NBUF).wait()

    return k(xt, table)


def kernel(x, table):
    xt = x.astype(jnp.int32).T
    out = _sc_gather(xt, table)
    return out.transpose(1, 0, 2)


# speedup vs baseline: 10.8149x; 1.0026x over previous
import functools

import jax
import jax.numpy as jnp
from jax import lax
from jax.experimental import pallas as pl
from jax.experimental.pallas import tpu as pltpu
from jax.experimental.pallas import tpu_sc as plsc

NUM_CLASSES = 100000
EMBED_DIM = 128
BATCH = 4096
SEQ = 50

NC = 2
NS = 16
NW = NC * NS

C = BATCH // NW
NBUF = 5
NITER = SEQ // NBUF


def _sc_gather(xt, table):
    mesh = plsc.VectorSubcoreMesh(
        core_axis_name="c", subcore_axis_name="s", num_cores=NC, num_subcores=NS
    )

    @functools.partial(
        pl.kernel,
        out_type=jax.ShapeDtypeStruct((SEQ, BATCH, EMBED_DIM), jnp.float32),
        mesh=mesh,
        scratch_types=[
            pltpu.VMEM((SEQ, C), jnp.int32),
            pltpu.VMEM((NBUF, C, EMBED_DIM), jnp.float32),
        ]
        + [pltpu.SemaphoreType.DMA] * (2 * NBUF),
        compiler_params=pltpu.CompilerParams(
            disable_bounds_checks=True, disable_semaphore_checks=True, skip_device_barrier=True
        ),
    )
    def k(xt_hbm, table_hbm, out_hbm, idx_v, rows_v, *sems):
        gsems, ssems = sems[:NBUF], sems[NBUF:]
        wid = lax.axis_index("s") * NC + lax.axis_index("c")
        col0 = wid * C
        pltpu.sync_copy(xt_hbm.at[:, pl.ds(col0, C)], idx_v)

        def gather(t, b):
            return pltpu.make_async_copy(
                table_hbm.at[idx_v.at[t]], rows_v.at[b], gsems[b]
            )

        def store(t, b):
            return pltpu.make_async_copy(
                rows_v.at[b], out_hbm.at[t, pl.ds(col0, C)], ssems[b]
            )

        for b in range(NBUF - 1):
            gather(b, b).start()

        def body(g, carry):
            for b in range(NBUF):
                t = g * NBUF + b
                pb = (b - 1) % NBUF
                gather(t, b).wait()
                store(t, b).start()
                if b == 0:
                    @pl.when(g > 0)
                    def _():
                        store(t - 1, pb).wait()

                else:
                    store(t - 1, pb).wait()

                @pl.when(t + NBUF - 1 < SEQ)
                def _():
                    gather(t + NBUF - 1, pb).start()

            return carry

        lax.fori_loop(0, NITER, body, 0)
        store(SEQ - 1, (SEQ - 1) % NBUF).wait()

    return k(xt, table)


def kernel(x, table):
    xt = x.astype(jnp.int32).T
    out = _sc_gather(xt, table)
    return out.transpose(1, 0, 2)
